# baseline (speedup 1.0000x reference)
import functools

import jax
import jax.numpy as jnp
from jax import lax
from jax.experimental import pallas as pl
from jax.experimental.pallas import tpu as pltpu
from jax.experimental.pallas import tpu_sc as plsc

NUM_EMB = 1000000
FEAT = 16
TOTAL = 16384 * 26
NC, NS, L = 2, 16, 16
NW = NC * NS

W_A = 1536
MAIN = 999936
NBLK = MAIN // W_A
KMAX = (NBLK + NW - 1) // NW
TAIL = NUM_EMB - MAIN

B_PER_W = TOTAL // NW
CHUNK = 1664
NCHUNK = B_PER_W // CHUNK


def _detile_body(tableT_hbm, tail_hbm, tableL_hbm,
                 rows0, rows1, trans0, trans1, tail_v,
                 sem_i0, sem_i1, sem_o0, sem_o1):
    w = lax.axis_index("s") * NC + lax.axis_index("c")
    iota16 = lax.iota(jnp.int32, 16)
    idx_f = [iota16 * FEAT + f for f in range(FEAT)]
    rows = (rows0, rows1)
    trans = (trans0, trans1)
    sem_i = (sem_i0, sem_i1)
    sem_o = (sem_o0, sem_o1)

    def fire_in(k):
        bi = k * NW + w

        @pl.when(bi < NBLK)
        def _():
            cb = pl.multiple_of(bi * W_A, 128)
            pltpu.async_copy(
                tableT_hbm.at[:, pl.ds(cb, W_A)], rows[k % 2], sem_i[k % 2]
            )

    fire_in(0)
    for k in range(KMAX):
        if k + 1 < KMAX:
            fire_in(k + 1)
        bi = k * NW + w

        @pl.when(bi < NBLK)
        def _():
            cb = pl.multiple_of(bi * W_A, 128)
            r_v, t_v = rows[k % 2], trans[k % 2]
            pltpu.make_async_copy(
                tableT_hbm.at[:, pl.ds(cb, W_A)], r_v, sem_i[k % 2]
            ).wait()
            if k >= 2:
                bi_p = (k - 2) * NW + w
                cb_p = pl.multiple_of(bi_p * W_A, 128)
                pltpu.make_async_copy(
                    t_v, tableL_hbm.at[pl.ds(cb_p * FEAT, W_A * FEAT)],
                    sem_o[k % 2],
                ).wait()

            for f in range(FEAT):
                @plsc.parallel_loop(0, W_A // 16, unroll=4)
                def grp(g):
                    v = r_v[f, pl.ds(g * 16, 16)]
                    plsc.store_scatter(
                        t_v.at[pl.ds(g * (16 * FEAT), 16 * FEAT)],
                        [idx_f[f]], v,
                    )
            pltpu.async_copy(
                t_v, tableL_hbm.at[pl.ds(cb * FEAT, W_A * FEAT)], sem_o[k % 2]
            )

    for k in (KMAX - 2, KMAX - 1):
        bi = k * NW + w

        @pl.when(bi < NBLK)
        def _():
            cb = pl.multiple_of(bi * W_A, 128)
            pltpu.make_async_copy(
                trans[k % 2], tableL_hbm.at[pl.ds(cb * FEAT, W_A * FEAT)],
                sem_o[k % 2],
            ).wait()

    @pl.when(w == 0)
    def _():
        pltpu.sync_copy(tail_hbm, tail_v)
        pltpu.sync_copy(tail_v, tableL_hbm.at[pl.ds(MAIN * FEAT, TAIL * FEAT)])


def _gather_body(table_hbm, idx_hbm, out_hbm, idx_v, rows_v, sem):
    wid = lax.axis_index("s") * NC + lax.axis_index("c")
    base = wid * B_PER_W
    pltpu.sync_copy(idx_hbm.at[pl.ds(base, B_PER_W)], idx_v)

    def add1(i, carry):
        for u in range(8):
            off = (i * 8 + u) * L
            idx_v[pl.ds(off, L)] = idx_v[pl.ds(off, L)] + 1
        return carry

    lax.fori_loop(0, B_PER_W // (8 * L), add1, 0)

    def chunk_body(c, carry):
        cbase = c * CHUNK
        pltpu.async_copy(
            table_hbm.at[idx_v.at[pl.ds(cbase, CHUNK)]], rows_v, sem
        ).wait()
        pltpu.sync_copy(rows_v, out_hbm.at[pl.ds(base + cbase, CHUNK)])
        return carry

    lax.fori_loop(0, NCHUNK, chunk_body, 0)


def _mesh():
    return plsc.VectorSubcoreMesh(core_axis_name="c", subcore_axis_name="s")


@jax.jit
def _launch(table, flat_idx):
    detile = functools.partial(
        pl.kernel,
        out_type=jax.ShapeDtypeStruct((NUM_EMB * FEAT,), jnp.float32),
        mesh=_mesh(),
        scratch_types=[
            pltpu.VMEM((FEAT, W_A), jnp.float32),
            pltpu.VMEM((FEAT, W_A), jnp.float32),
            pltpu.VMEM((W_A * FEAT,), jnp.float32),
            pltpu.VMEM((W_A * FEAT,), jnp.float32),
            pltpu.VMEM((TAIL * FEAT,), jnp.float32),
            pltpu.SemaphoreType.DMA,
            pltpu.SemaphoreType.DMA,
            pltpu.SemaphoreType.DMA,
            pltpu.SemaphoreType.DMA,
        ],
        compiler_params=pltpu.CompilerParams(
            use_tc_tiling_on_sc=True, needs_layout_passes=False,
            disable_bounds_checks=True,
        ),
    )(_detile_body)
    tail = lax.slice(table, (MAIN, 0), (NUM_EMB, FEAT)).reshape(-1)
    table_lin = detile(table.T, tail).reshape(NUM_EMB, FEAT)

    gather = functools.partial(
        pl.kernel,
        out_type=jax.ShapeDtypeStruct((TOTAL, FEAT), jnp.float32),
        mesh=_mesh(),
        scratch_types=[
            pltpu.VMEM((B_PER_W,), jnp.int32),
            pltpu.VMEM((CHUNK, FEAT), jnp.float32),
            pltpu.SemaphoreType.DMA,
        ],
        compiler_params=pltpu.CompilerParams(
            use_tc_tiling_on_sc=False, needs_layout_passes=False
        ),
    )(_gather_body)
    return gather(table_lin, flat_idx)


def kernel(x, embedding_table):
    flat_idx = x.reshape(-1)
    out = _launch(embedding_table, flat_idx)
    return out.reshape(x.shape[0], x.shape[1] * FEAT)


# speedup vs baseline: 1.4072x; 1.4072x over previous
import functools

import jax
import jax.numpy as jnp
from jax import lax
from jax.experimental import pallas as pl
from jax.experimental.pallas import tpu as pltpu
from jax.experimental.pallas import tpu_sc as plsc

NUM_EMB = 1000000
FEAT = 16
TOTAL = 16384 * 26
NC, NS, L = 2, 16, 16
NW = NC * NS

W_A = 1536
MAIN = 999936
NBLK = MAIN // W_A
KMAX = (NBLK + NW - 1) // NW
TAIL = NUM_EMB - MAIN

B_PER_W = TOTAL // NW
CHUNK = 1664
NCHUNK = B_PER_W // CHUNK


def _detile_body(tableT_hbm, tail_hbm, tableL_hbm,
                 rows0, rows1, trans0, trans1, tail_v,
                 sem_i0, sem_i1, sem_o0, sem_o1):
    w = lax.axis_index("s") * NC + lax.axis_index("c")
    iota16 = lax.iota(jnp.int32, 16)
    idx_f = [iota16 * FEAT + f for f in range(FEAT)]
    rows = (rows0, rows1)
    trans = (trans0, trans1)
    sem_i = (sem_i0, sem_i1)
    sem_o = (sem_o0, sem_o1)

    def fire_in(k):
        bi = k * NW + w

        @pl.when(bi < NBLK)
        def _():
            cb = pl.multiple_of(bi * W_A, 128)
            pltpu.async_copy(
                tableT_hbm.at[:, pl.ds(cb, W_A)], rows[k % 2], sem_i[k % 2]
            )

    fire_in(0)
    for k in range(KMAX):
        if k + 1 < KMAX:
            fire_in(k + 1)
        bi = k * NW + w

        @pl.when(bi < NBLK)
        def _():
            cb = pl.multiple_of(bi * W_A, 128)
            r_v, t_v = rows[k % 2], trans[k % 2]
            pltpu.make_async_copy(
                tableT_hbm.at[:, pl.ds(cb, W_A)], r_v, sem_i[k % 2]
            ).wait()
            if k >= 2:
                bi_p = (k - 2) * NW + w
                cb_p = pl.multiple_of(bi_p * W_A, 128)
                pltpu.make_async_copy(
                    t_v, tableL_hbm.at[pl.ds(cb_p * FEAT, W_A * FEAT)],
                    sem_o[k % 2],
                ).wait()

            @plsc.parallel_loop(0, W_A // 16, unroll=2)
            def grp(g):
                base = g * (16 * FEAT)
                for f in range(FEAT):
                    v = r_v[f, pl.ds(g * 16, 16)]
                    plsc.store_scatter(
                        t_v.at[pl.ds(base, 16 * FEAT)], [idx_f[f]], v
                    )
            pltpu.async_copy(
                t_v, tableL_hbm.at[pl.ds(cb * FEAT, W_A * FEAT)], sem_o[k % 2]
            )

    for k in (KMAX - 2, KMAX - 1):
        bi = k * NW + w

        @pl.when(bi < NBLK)
        def _():
            cb = pl.multiple_of(bi * W_A, 128)
            pltpu.make_async_copy(
                trans[k % 2], tableL_hbm.at[pl.ds(cb * FEAT, W_A * FEAT)],
                sem_o[k % 2],
            ).wait()

    @pl.when(w == 0)
    def _():
        pltpu.sync_copy(tail_hbm, tail_v)
        pltpu.sync_copy(tail_v, tableL_hbm.at[pl.ds(MAIN * FEAT, TAIL * FEAT)])


def _gather_body(table_hbm, idx_hbm, out_hbm, idx_v,
                 rv0, rv1, rv2, sg0, sg1, so0, so1):
    wid = lax.axis_index("s") * NC + lax.axis_index("c")
    base = wid * B_PER_W
    rv = (rv0, rv1, rv2)
    sg = (sg0, sg1)
    so = (so0, so1)
    pltpu.sync_copy(idx_hbm.at[pl.ds(base, B_PER_W)], idx_v)

    @plsc.parallel_loop(0, B_PER_W // (8 * L), unroll=2)
    def add1(i):
        for u in range(8):
            off = (i * 8 + u) * L
            idx_v[pl.ds(off, L)] = idx_v[pl.ds(off, L)] + 1

    def gref(c):
        return table_hbm.at[idx_v.at[pl.ds(c * CHUNK, CHUNK)]]

    def oref(c):
        return out_hbm.at[pl.ds(base + c * CHUNK, CHUNK)]

    pltpu.async_copy(gref(0), rv[0], sg[0])
    for c in range(NCHUNK):
        if c >= 2:
            pltpu.make_async_copy(rv[(c - 2) % 3], oref(c - 2), so[c % 2]).wait()
        if c + 1 < NCHUNK:
            pltpu.async_copy(gref(c + 1), rv[(c + 1) % 3], sg[(c + 1) % 2])
        pltpu.make_async_copy(gref(c), rv[c % 3], sg[c % 2]).wait()
        pltpu.async_copy(rv[c % 3], oref(c), so[c % 2])
    for c in (NCHUNK - 2, NCHUNK - 1):
        pltpu.make_async_copy(rv[c % 3], oref(c), so[c % 2]).wait()


def _mesh():
    return plsc.VectorSubcoreMesh(core_axis_name="c", subcore_axis_name="s")


@jax.jit
def _launch(table, flat_idx):
    detile = functools.partial(
        pl.kernel,
        out_type=jax.ShapeDtypeStruct((NUM_EMB * FEAT,), jnp.float32),
        mesh=_mesh(),
        scratch_types=[
            pltpu.VMEM((FEAT, W_A), jnp.float32),
            pltpu.VMEM((FEAT, W_A), jnp.float32),
            pltpu.VMEM((W_A * FEAT,), jnp.float32),
            pltpu.VMEM((W_A * FEAT,), jnp.float32),
            pltpu.VMEM((TAIL * FEAT,), jnp.float32),
            pltpu.SemaphoreType.DMA,
            pltpu.SemaphoreType.DMA,
            pltpu.SemaphoreType.DMA,
            pltpu.SemaphoreType.DMA,
        ],
        compiler_params=pltpu.CompilerParams(
            use_tc_tiling_on_sc=True, needs_layout_passes=False,
            disable_bounds_checks=True,
        ),
    )(_detile_body)
    tail = lax.slice(table, (MAIN, 0), (NUM_EMB, FEAT)).reshape(-1)
    table_lin = detile(table.T, tail).reshape(NUM_EMB, FEAT)

    gather = functools.partial(
        pl.kernel,
        out_type=jax.ShapeDtypeStruct((TOTAL, FEAT), jnp.float32),
        mesh=_mesh(),
        scratch_types=[
            pltpu.VMEM((B_PER_W,), jnp.int32),
            pltpu.VMEM((CHUNK, FEAT), jnp.float32),
            pltpu.VMEM((CHUNK, FEAT), jnp.float32),
            pltpu.VMEM((CHUNK, FEAT), jnp.float32),
            pltpu.SemaphoreType.DMA,
            pltpu.SemaphoreType.DMA,
            pltpu.SemaphoreType.DMA,
            pltpu.SemaphoreType.DMA,
        ],
        compiler_params=pltpu.CompilerParams(
            use_tc_tiling_on_sc=False, needs_layout_passes=False
        ),
    )(_gather_body)
    return gather(table_lin, flat_idx)


def kernel(x, embedding_table):
    flat_idx = x.reshape(-1)
    out = _launch(embedding_table, flat_idx)
    return out.reshape(x.shape[0], x.shape[1] * FEAT)


# speedup vs baseline: 1.4247x; 1.0124x over previous
import functools

import jax
import jax.numpy as jnp
from jax import lax
from jax.experimental import pallas as pl
from jax.experimental.pallas import tpu as pltpu
from jax.experimental.pallas import tpu_sc as plsc

NUM_EMB = 1000000
FEAT = 16
TOTAL = 16384 * 26
NC, NS, L = 2, 16, 16
NW = NC * NS

W_A = 1792
MAIN = 999936
NBLK = MAIN // W_A
KMAX = (NBLK + NW - 1) // NW
TAIL = NUM_EMB - MAIN

B_PER_W = TOTAL // NW
CHUNK = 1664
NCHUNK = B_PER_W // CHUNK


def _detile_body(tableT_hbm, tail_hbm, tableL_hbm,
                 rows0, rows1, trans0, trans1, tail_v,
                 sem_i0, sem_i1, sem_o0, sem_o1):
    w = lax.axis_index("s") * NC + lax.axis_index("c")
    iota16 = lax.iota(jnp.int32, 16)
    idx_f = [iota16 * FEAT + f for f in range(FEAT)]
    rows = (rows0, rows1)
    trans = (trans0, trans1)
    sem_i = (sem_i0, sem_i1)
    sem_o = (sem_o0, sem_o1)

    def fire_in(k):
        bi = k * NW + w

        @pl.when(bi < NBLK)
        def _():
            cb = pl.multiple_of(bi * W_A, 128)
            pltpu.async_copy(
                tableT_hbm.at[:, pl.ds(cb, W_A)], rows[k % 2], sem_i[k % 2]
            )

    fire_in(0)
    for k in range(KMAX):
        if k + 1 < KMAX:
            fire_in(k + 1)
        bi = k * NW + w

        @pl.when(bi < NBLK)
        def _():
            cb = pl.multiple_of(bi * W_A, 128)
            r_v, t_v = rows[k % 2], trans[k % 2]
            pltpu.make_async_copy(
                tableT_hbm.at[:, pl.ds(cb, W_A)], r_v, sem_i[k % 2]
            ).wait()
            if k >= 2:
                bi_p = (k - 2) * NW + w
                cb_p = pl.multiple_of(bi_p * W_A, 128)
                pltpu.make_async_copy(
                    t_v, tableL_hbm.at[pl.ds(cb_p * FEAT, W_A * FEAT)],
                    sem_o[k % 2],
                ).wait()

            @plsc.parallel_loop(0, W_A // 16, unroll=2)
            def grp(g):
                base = g * (16 * FEAT)
                for f in range(FEAT):
                    v = r_v[f, pl.ds(g * 16, 16)]
                    plsc.store_scatter(
                        t_v.at[pl.ds(base, 16 * FEAT)], [idx_f[f]], v
                    )
            pltpu.async_copy(
                t_v, tableL_hbm.at[pl.ds(cb * FEAT, W_A * FEAT)], sem_o[k % 2]
            )

    for k in (KMAX - 2, KMAX - 1):
        bi = k * NW + w

        @pl.when(bi < NBLK)
        def _():
            cb = pl.multiple_of(bi * W_A, 128)
            pltpu.make_async_copy(
                trans[k % 2], tableL_hbm.at[pl.ds(cb * FEAT, W_A * FEAT)],
                sem_o[k % 2],
            ).wait()

    @pl.when(w == 0)
    def _():
        pltpu.sync_copy(tail_hbm, tail_v)
        pltpu.sync_copy(tail_v, tableL_hbm.at[pl.ds(MAIN * FEAT, TAIL * FEAT)])


def _gather_body(table_hbm, idx_hbm, out_hbm, idx_v,
                 rv0, rv1, rv2, sg0, sg1, so0, so1):
    wid = lax.axis_index("s") * NC + lax.axis_index("c")
    base = wid * B_PER_W
    rv = (rv0, rv1, rv2)
    sg = (sg0, sg1)
    so = (so0, so1)
    pltpu.sync_copy(idx_hbm.at[pl.ds(base, B_PER_W)], idx_v)

    @plsc.parallel_loop(0, B_PER_W // (8 * L), unroll=2)
    def add1(i):
        for u in range(8):
            off = (i * 8 + u) * L
            idx_v[pl.ds(off, L)] = idx_v[pl.ds(off, L)] + 1

    def gref(c):
        return table_hbm.at[idx_v.at[pl.ds(c * CHUNK, CHUNK)]]

    def oref(c):
        return out_hbm.at[pl.ds(base + c * CHUNK, CHUNK)]

    pltpu.async_copy(gref(0), rv[0], sg[0])
    for c in range(NCHUNK):
        if c >= 2:
            pltpu.make_async_copy(rv[(c - 2) % 3], oref(c - 2), so[c % 2]).wait()
        if c + 1 < NCHUNK:
            pltpu.async_copy(gref(c + 1), rv[(c + 1) % 3], sg[(c + 1) % 2])
        pltpu.make_async_copy(gref(c), rv[c % 3], sg[c % 2]).wait()
        pltpu.async_copy(rv[c % 3], oref(c), so[c % 2])
    for c in (NCHUNK - 2, NCHUNK - 1):
        pltpu.make_async_copy(rv[c % 3], oref(c), so[c % 2]).wait()


def _mesh():
    return plsc.VectorSubcoreMesh(core_axis_name="c", subcore_axis_name="s")


@jax.jit
def _launch(table, flat_idx):
    detile = functools.partial(
        pl.kernel,
        out_type=jax.ShapeDtypeStruct((NUM_EMB * FEAT,), jnp.float32),
        mesh=_mesh(),
        scratch_types=[
            pltpu.VMEM((FEAT, W_A), jnp.float32),
            pltpu.VMEM((FEAT, W_A), jnp.float32),
            pltpu.VMEM((W_A * FEAT,), jnp.float32),
            pltpu.VMEM((W_A * FEAT,), jnp.float32),
            pltpu.VMEM((TAIL * FEAT,), jnp.float32),
            pltpu.SemaphoreType.DMA,
            pltpu.SemaphoreType.DMA,
            pltpu.SemaphoreType.DMA,
            pltpu.SemaphoreType.DMA,
        ],
        compiler_params=pltpu.CompilerParams(
            use_tc_tiling_on_sc=True, needs_layout_passes=False,
            disable_bounds_checks=True,
        ),
    )(_detile_body)
    tail = lax.slice(table, (MAIN, 0), (NUM_EMB, FEAT)).reshape(-1)
    table_lin = detile(table.T, tail).reshape(NUM_EMB, FEAT)

    gather = functools.partial(
        pl.kernel,
        out_type=jax.ShapeDtypeStruct((TOTAL, FEAT), jnp.float32),
        mesh=_mesh(),
        scratch_types=[
            pltpu.VMEM((B_PER_W,), jnp.int32),
            pltpu.VMEM((CHUNK, FEAT), jnp.float32),
            pltpu.VMEM((CHUNK, FEAT), jnp.float32),
            pltpu.VMEM((CHUNK, FEAT), jnp.float32),
            pltpu.SemaphoreType.DMA,
            pltpu.SemaphoreType.DMA,
            pltpu.SemaphoreType.DMA,
            pltpu.SemaphoreType.DMA,
        ],
        compiler_params=pltpu.CompilerParams(
            use_tc_tiling_on_sc=False, needs_layout_passes=False
        ),
    )(_gather_body)
    return gather(table_lin, flat_idx)


def kernel(x, embedding_table):
    flat_idx = x.reshape(-1)
    out = _launch(embedding_table, flat_idx)
    return out.reshape(x.shape[0], x.shape[1] * FEAT)
